# Initial kernel scaffold; baseline (speedup 1.0000x reference)
import jax
import jax.numpy as jnp
from jax.experimental import pallas as pl


def kernel(X, table):
    raise NotImplementedError("write your pallas kernel here")


# speedup vs baseline: 1.0987x; 1.0987x over previous
import math
import functools

import jax
import jax.numpy as jnp
from jax import lax
from jax.experimental import pallas as pl
from jax.experimental.pallas import tpu as pltpu
from jax.experimental.pallas import tpu_sc as plsc

D = 32
L = 16
NC = 2
NS = 16
NW = NC * NS
SCALE = math.sqrt(D)


def _make_kernel(B):
    assert B % NW == 0
    b_per_w = B // NW
    CH = 1600
    while b_per_w % CH:
        CH //= 2
    n_chunks = b_per_w // CH

    mesh = plsc.VectorSubcoreMesh(core_axis_name="c", subcore_axis_name="s")

    @functools.partial(
        pl.kernel,
        out_type=jax.ShapeDtypeStruct((B, D), jnp.float32),
        mesh=mesh,
        scratch_types=[
            pltpu.VMEM((b_per_w,), jnp.int32),
            pltpu.VMEM((CH, D), jnp.float32),
            pltpu.VMEM((CH, D), jnp.float32),
            pltpu.SemaphoreType.DMA,
            pltpu.SemaphoreType.DMA,
            pltpu.SemaphoreType.DMA,
            pltpu.SemaphoreType.DMA,
        ],
        compiler_params=pltpu.CompilerParams(use_tc_tiling_on_sc=False),
    )
    def emb_kernel(idx_hbm, table_hbm, out_hbm, idx_v, rows0, rows1,
                   gsem0, gsem1, osem0, osem1):
        wid = lax.axis_index("s") * NC + lax.axis_index("c")
        base = wid * b_per_w
        pltpu.sync_copy(idx_hbm.at[pl.ds(base, b_per_w)], idx_v)

        rows = (rows0, rows1)
        gsems = (gsem0, gsem1)
        osems = (osem0, osem1)

        def gather(c, buf):
            return pltpu.async_copy(
                table_hbm.at[idx_v.at[pl.ds(c * CH, CH)]],
                rows[buf], gsems[buf])

        def scale(buf):
            r = rows[buf]

            def body(i, _):
                r[i, pl.ds(0, L)] = r[i, pl.ds(0, L)] * SCALE
                r[i, pl.ds(L, L)] = r[i, pl.ds(L, L)] * SCALE
                return 0

            lax.fori_loop(0, CH, body, 0, unroll=8)

        def store(c, buf):
            return pltpu.async_copy(
                rows[buf], out_hbm.at[pl.ds(base + c * CH, CH)], osems[buf])

        pending_gather = [None, None]
        pending_store = [None, None]

        pending_gather[0] = gather(0, 0)
        for c in range(n_chunks):
            buf = c % 2
            nxt = 1 - buf
            if c + 1 < n_chunks:
                if pending_store[nxt] is not None:
                    pending_store[nxt].wait()
                    pending_store[nxt] = None
                pending_gather[nxt] = gather(c + 1, nxt)
            pending_gather[buf].wait()
            scale(buf)
            pending_store[buf] = store(c, buf)
        for b in range(2):
            if pending_store[b] is not None:
                pending_store[b].wait()

    return emb_kernel


def kernel(X, table):
    B_, H_ = X.shape
    B = B_ * H_
    idx = X.reshape(B).astype(jnp.int32)
    out = _make_kernel(B)(idx, table)
    return out.reshape(B_, H_, D)


# speedup vs baseline: 1.5462x; 1.4073x over previous
import math
import functools

import jax
import jax.numpy as jnp
from jax import lax
from jax.experimental import pallas as pl
from jax.experimental.pallas import tpu as pltpu
from jax.experimental.pallas import tpu_sc as plsc

D = 32
L = 16
NC = 2
NS = 16
NW = NC * NS
SCALE = math.sqrt(D)

H = 50
B = 16384
BW = B // NW
NT = BW // 128
DT = D // 8

_mesh = plsc.VectorSubcoreMesh(core_axis_name="c", subcore_axis_name="s")


@functools.partial(
    pl.kernel,
    out_type=jax.ShapeDtypeStruct((H, DT, B // 128, 8, 128), jnp.float32),
    mesh=_mesh,
    scratch_types=[
        pltpu.VMEM((H, BW), jnp.int32),
        pltpu.VMEM((BW, D), jnp.float32),
        pltpu.VMEM((BW, D), jnp.float32),
        pltpu.VMEM((DT, NT, 8, 128), jnp.float32),
        pltpu.VMEM((DT, NT, 8, 128), jnp.float32),
        pltpu.SemaphoreType.DMA,
        pltpu.SemaphoreType.DMA,
        pltpu.SemaphoreType.DMA,
        pltpu.SemaphoreType.DMA,
    ],
    compiler_params=pltpu.CompilerParams(
        use_tc_tiling_on_sc=False, needs_layout_passes=False),
)
def _emb_kernel(xt_hbm, table_hbm, out_hbm, idx_all, rows0, rows1,
                tbuf0, tbuf1, gsem0, gsem1, osem0, osem1):
    wid = lax.axis_index("s") * NC + lax.axis_index("c")
    b0 = wid * BW
    bt0 = wid * NT

    rows = (rows0, rows1)
    tbufs = (tbuf0, tbuf1)
    gsems = (gsem0, gsem1)
    osems = (osem0, osem1)

    pltpu.sync_copy(xt_hbm.at[:, pl.ds(b0, BW)], idx_all)

    def issue_gather(h, buf):
        pltpu.async_copy(
            table_hbm.at[idx_all.at[h]], rows[buf], gsems[buf])

    def wait_gather(h, buf):
        pltpu.make_async_copy(
            table_hbm.at[idx_all.at[h]], rows[buf], gsems[buf]).wait()

    def wait_out(h, buf):
        pltpu.make_async_copy(
            tbufs[buf], out_hbm.at[h, :, pl.ds(bt0, NT)], osems[buf]).wait()

    iota16 = jax.lax.iota(jnp.int32, L)
    dvecs = [jnp.full((L,), d, jnp.int32) for d in range(D)]

    def compute(h, buf):
        r = rows[buf]
        tb = tbufs[buf]

        def inner(i2, _):
            lvec = i2 * L + iota16
            t = i2 // 8
            j16 = (i2 % 8) * L
            for d in range(D):
                dt, s = divmod(d, 8)
                g = plsc.load_gather(r, [lvec, dvecs[d]])
                tb[dt, t, s, pl.ds(j16, L)] = g * SCALE
            return 0

        lax.fori_loop(0, BW // L, inner, 0)
        pltpu.async_copy(tb, out_hbm.at[h, :, pl.ds(bt0, NT)], osems[buf])

    issue_gather(0, 0)
    issue_gather(1, 1)

    def step(i, _):
        h0 = 2 * i
        h1 = 2 * i + 1

        wait_gather(h0, 0)

        @pl.when(i >= 1)
        def _():
            wait_out(h0, 0)

        compute(h0, 0)

        @pl.when(h0 + 2 < H)
        def _():
            issue_gather(h0 + 2, 0)

        wait_gather(h1, 1)

        @pl.when(i >= 1)
        def _():
            wait_out(h1, 1)

        compute(h1, 1)

        @pl.when(h1 + 2 < H)
        def _():
            issue_gather(h1 + 2, 1)

        return 0

    lax.fori_loop(0, H // 2, step, 0)
    wait_out(H - 2, 0)
    wait_out(H - 1, 1)


def kernel(X, table):
    xt = X.T.astype(jnp.int32)
    outp = _emb_kernel(xt, table)
    out = outp.transpose(0, 1, 3, 2, 4).reshape(H, D, B).transpose(2, 0, 1)
    return out


# speedup vs baseline: 2.0139x; 1.3025x over previous
import math
import functools

import jax
import jax.numpy as jnp
from jax import lax
from jax.experimental import pallas as pl
from jax.experimental.pallas import tpu as pltpu
from jax.experimental.pallas import tpu_sc as plsc

D = 32
L = 16
NC = 2
NS = 16
NW = NC * NS
SCALE = math.sqrt(D)

H = 50
B = 16384
BW = B // NW
NT = BW // 128
DT = D // 8

_mesh = plsc.VectorSubcoreMesh(core_axis_name="c", subcore_axis_name="s")


@functools.partial(
    pl.kernel,
    out_type=jax.ShapeDtypeStruct((H, DT, B // 128, 8, 128), jnp.float32),
    mesh=_mesh,
    scratch_types=[
        pltpu.VMEM((H, BW), jnp.int32),
        pltpu.VMEM((BW, D), jnp.float32),
        pltpu.VMEM((BW, D), jnp.float32),
        pltpu.VMEM((DT, NT, 8, 128), jnp.float32),
        pltpu.VMEM((DT, NT, 8, 128), jnp.float32),
        pltpu.SemaphoreType.DMA,
        pltpu.SemaphoreType.DMA,
        pltpu.SemaphoreType.DMA,
        pltpu.SemaphoreType.DMA,
    ],
    compiler_params=pltpu.CompilerParams(
        use_tc_tiling_on_sc=False, needs_layout_passes=False),
)
def _emb_kernel(xt_hbm, table_hbm, out_hbm, idx_all, rows0, rows1,
                tbuf0, tbuf1, gsem0, gsem1, osem0, osem1):
    wid = lax.axis_index("s") * NC + lax.axis_index("c")
    b0 = wid * BW
    bt0 = wid * NT

    rows = (rows0, rows1)
    tbufs = (tbuf0, tbuf1)
    gsems = (gsem0, gsem1)
    osems = (osem0, osem1)

    pltpu.sync_copy(xt_hbm.at[:, pl.ds(b0, BW)], idx_all)

    def issue_gather(h, buf):
        pltpu.async_copy(
            table_hbm.at[idx_all.at[h]], rows[buf], gsems[buf])

    def wait_gather(h, buf):
        pltpu.make_async_copy(
            table_hbm.at[idx_all.at[h]], rows[buf], gsems[buf]).wait()

    def wait_out(h, buf):
        pltpu.make_async_copy(
            tbufs[buf], out_hbm.at[h, :, pl.ds(bt0, NT)], osems[buf]).wait()

    iota16 = jax.lax.iota(jnp.int32, L)

    def compute(h, buf):
        r = rows[buf]
        tb = tbufs[buf]

        @plsc.parallel_loop(0, BW // L, unroll=2)
        def _(i2):
            lvec = i2 * L + iota16
            t = i2 // 8
            j16 = (i2 % 8) * L
            for d in range(D):
                dt, s = divmod(d, 8)
                g = plsc.load_gather(r, [lvec, jnp.full((L,), d, jnp.int32)])
                tb[dt, t, s, pl.ds(j16, L)] = g * SCALE

        pltpu.async_copy(tb, out_hbm.at[h, :, pl.ds(bt0, NT)], osems[buf])

    issue_gather(0, 0)
    issue_gather(1, 1)

    def step(i, _):
        h0 = 2 * i
        h1 = 2 * i + 1

        wait_gather(h0, 0)

        @pl.when(i >= 1)
        def _():
            wait_out(h0, 0)

        compute(h0, 0)

        @pl.when(h0 + 2 < H)
        def _():
            issue_gather(h0 + 2, 0)

        wait_gather(h1, 1)

        @pl.when(i >= 1)
        def _():
            wait_out(h1, 1)

        compute(h1, 1)

        @pl.when(h1 + 2 < H)
        def _():
            issue_gather(h1 + 2, 1)

        return 0

    lax.fori_loop(0, H // 2, step, 0)
    wait_out(H - 2, 0)
    wait_out(H - 1, 1)


def kernel(X, table):
    xt = X.T.astype(jnp.int32)
    outp = _emb_kernel(xt, table)
    out = outp.transpose(0, 1, 3, 2, 4).reshape(H, D, B).transpose(2, 0, 1)
    return out
